# baseline (speedup 1.0000x reference)
import functools

import jax
import jax.numpy as jnp
from jax import lax
from jax.experimental import pallas as pl
from jax.experimental.pallas import tpu as pltpu
from jax.experimental.pallas import tpu_sc as plsc

N = 10000
E = 320000
H = 128
LAYERS = 3
NG = 64
OUT = 10
NP = 10240
NBUK = 64
BR = 160
NW = 32
EC = E // NW
EP = E + 256
DUMP = 5120
K = 128
NEG = -3.0e38

_i16 = lambda: lax.iota(jnp.int32, 16)


def _mesh():
    return plsc.VectorSubcoreMesh(core_axis_name="c", subcore_axis_name="s")


def _p1(dst):
    @functools.partial(
        pl.kernel,
        out_type=[jax.ShapeDtypeStruct((NW, 1024), jnp.int32),
                  jax.ShapeDtypeStruct((2 * NP,), jnp.float32)],
        mesh=_mesh(),
        compiler_params=pltpu.CompilerParams(needs_layout_passes=False),
        scratch_types=[
            pltpu.VMEM((2048,), jnp.int32),
            pltpu.VMEM((16, 128), jnp.int32),
            pltpu.VMEM((128,), jnp.float32),
            pltpu.VMEM((1040,), jnp.int32),
            pltpu.VMEM((640,), jnp.float32),
            pltpu.VMEM_SHARED((NP + 16,), jnp.float32),
            pltpu.SemaphoreType.DMA,
        ],
    )
    def k(dst_ref, hist_ref, degp_ref, draw, d2, onesb, histv, zb, deg_sh,
          sem):
        c = lax.axis_index("c")
        s = lax.axis_index("s")
        wid = c * 16 + s
        zf = jnp.zeros((16,), jnp.float32)
        zi = jnp.zeros((16,), jnp.int32)
        one_i = jnp.full((16,), 1, jnp.int32)
        onef = jnp.full((16,), 1.0, jnp.float32)
        lane = _i16()

        def z1(i, _):
            histv[pl.ds(i * 16, 16)] = zi
            return 0
        lax.fori_loop(0, 65, z1, 0)

        def z2(i, _):
            zb[pl.ds(i * 16, 16)] = zf
            return 0
        lax.fori_loop(0, 40, z2, 0)
        for i in range(8):
            onesb[pl.ds(i * 16, 16)] = onef
        pltpu.sync_copy(zb, deg_sh.at[pl.ds(s * 640, 640)])

        @pl.when(s == 0)
        def _():
            zb16 = zb.at[pl.ds(0, 16)]
            pltpu.sync_copy(zb16, deg_sh.at[pl.ds(NP, 16)])

        plsc.subcore_barrier()

        for ch in range(5):
            sz = 2048 if ch < 4 else 1808
            g_n = sz // 16
            pltpu.sync_copy(dst_ref.at[pl.ds(wid * EC + ch * 2048, sz)],
                            draw.at[pl.ds(0, sz)])
            if ch == 4:
                dumpv = jnp.full((16,), NP, jnp.int32)
                def zt(i, _):
                    for cc in range(8):
                        d2[i, pl.ds(cc * 16, 16)] = dumpv
                    return 0
                lax.fori_loop(0, 16, zt, 0)

            def body(g, _):
                dv = draw[pl.ds(g * 16, 16)]
                b = jnp.right_shift(dv * 13108, 21)
                idx = b * 16 + lane
                plsc.addupdate_scatter(histv, [idx], one_i)
                d2[g // 8, pl.ds((g % 8) * 16, 16)] = dv
                return 0
            lax.fori_loop(0, g_n, body, 0)

            cps = [pltpu.async_copy(onesb, deg_sh.at[d2.at[i]], sem,
                                    add=True) for i in range(16)]
            for cp in cps:
                cp.wait()

        plsc.subcore_barrier()
        pltpu.sync_copy(histv.at[pl.ds(0, 1024)], hist_ref.at[wid])
        pltpu.sync_copy(deg_sh.at[pl.ds(s * 640, 640)],
                        degp_ref.at[pl.ds(c * NP + s * 640, 640)])

    return k(dst)


def _off_body(hist_ref, degp_ref, starts_ref, deg_ref):
    cnt = hist_ref[...].astype(jnp.float32)
    ii = lax.broadcasted_iota(jnp.int32, (1024, 1024), 0)
    jj = lax.broadcasted_iota(jnp.int32, (1024, 1024), 1)
    m_l = ((ii // 16 == jj // 16) & ((ii % 16) < (jj % 16))).astype(
        jnp.float32)
    lstart = jnp.dot(cnt, m_l, preferred_element_type=jnp.float32, precision=lax.Precision.HIGHEST)
    r64 = lax.broadcasted_iota(jnp.int32, (1024, 64), 0) // 16
    c64 = lax.broadcasted_iota(jnp.int32, (1024, 64), 1)
    msum = (r64 == c64).astype(jnp.float32)
    tbc = jnp.dot(cnt, msum, preferred_element_type=jnp.float32, precision=lax.Precision.HIGHEST)
    aw = lax.broadcasted_iota(jnp.int32, (32, 32), 1)
    ar = lax.broadcasted_iota(jnp.int32, (32, 32), 0)
    a32 = (aw < ar).astype(jnp.float32)
    wstart = jnp.dot(a32, tbc, preferred_element_type=jnp.float32, precision=lax.Precision.HIGHEST)
    ones32 = jnp.ones((1, 32), jnp.float32)
    bcnt = jnp.dot(ones32, tbc, preferred_element_type=jnp.float32, precision=lax.Precision.HIGHEST)
    li = lax.broadcasted_iota(jnp.int32, (64, 64), 0)
    lj = lax.broadcasted_iota(jnp.int32, (64, 64), 1)
    lt64 = (li < lj).astype(jnp.float32)
    bs = jnp.dot(bcnt, lt64, preferred_element_type=jnp.float32, precision=lax.Precision.HIGHEST)
    e64r = lax.broadcasted_iota(jnp.int32, (64, 1024), 0)
    e64c = lax.broadcasted_iota(jnp.int32, (64, 1024), 1) // 16
    e64 = (e64r == e64c).astype(jnp.float32)
    bs_e = jnp.dot(bs, e64, preferred_element_type=jnp.float32, precision=lax.Precision.HIGHEST)
    ws_e = jnp.dot(wstart, e64, preferred_element_type=jnp.float32, precision=lax.Precision.HIGHEST)
    starts_ref[...] = (bs_e + ws_e + lstart).astype(jnp.int32)
    dp = degp_ref[...].reshape(2, NP)
    degsum = dp[0:1, :] + dp[1:2, :]
    deg_ref[...] = degsum.reshape(NP // 256, 1, 256)


def _offsets(hist, degp):
    return pl.pallas_call(
        _off_body,
        out_shape=[jax.ShapeDtypeStruct((NW, 1024), jnp.int32),
                   jax.ShapeDtypeStruct((NP // 256, 1, 256), jnp.float32)],
    )(hist, degp)


def _p2(src, dst, starts):
    @functools.partial(
        pl.kernel,
        out_type=[jax.ShapeDtypeStruct((EP,), jnp.int32),
                  jax.ShapeDtypeStruct((EP,), jnp.int32)],
        mesh=_mesh(),
        compiler_params=pltpu.CompilerParams(needs_layout_passes=False),
        scratch_types=[
            pltpu.VMEM((1040,), jnp.int32),
            pltpu.VMEM((2048,), jnp.int32),
            pltpu.VMEM((2048,), jnp.int32),
            pltpu.VMEM((16, 128), jnp.int32),
            pltpu.VMEM((16, 128), jnp.int32),
            pltpu.VMEM((128,), jnp.int32),
            pltpu.SemaphoreType.DMA,
            pltpu.SemaphoreType.DMA,
        ],
    )
    def k(src_ref, dst_ref, starts_ref, srcp_ref, dstlp_ref, offs, sraw,
          draw, pos2, dl2, padv, sem, sem2):
        c = lax.axis_index("c")
        s = lax.axis_index("s")
        wid = c * 16 + s
        lane = _i16()

        @pl.when(wid == 0)
        def _():
            pv = jnp.full((16,), NP - 1, jnp.int32)
            zv = jnp.zeros((16,), jnp.int32)
            for i in range(8):
                padv[pl.ds(i * 16, 16)] = pv
            pltpu.sync_copy(padv, srcp_ref.at[pl.ds(E, 128)])
            for i in range(8):
                padv[pl.ds(i * 16, 16)] = zv
            pltpu.sync_copy(padv, dstlp_ref.at[pl.ds(E, 128)])

        pltpu.sync_copy(starts_ref.at[wid], offs.at[pl.ds(0, 1024)])

        for ch in range(5):
            sz = 2048 if ch < 4 else 1808
            g_n = sz // 16
            base = wid * EC + ch * 2048
            pltpu.sync_copy(src_ref.at[pl.ds(base, sz)],
                            sraw.at[pl.ds(0, sz)])
            pltpu.sync_copy(dst_ref.at[pl.ds(base, sz)],
                            draw.at[pl.ds(0, sz)])
            if ch == 4:
                def zt(i, _):
                    dump = jnp.full((16,), E + 128, jnp.int32) + lane
                    for cc in range(8):
                        pos2[i, pl.ds(cc * 16, 16)] = dump
                    return 0
                lax.fori_loop(0, 16, zt, 0)

            def body(g, _):
                dv = draw[pl.ds(g * 16, 16)]
                b = jnp.right_shift(dv * 13108, 21)
                idx = b * 16 + lane
                cur = plsc.load_gather(offs, [idx])
                plsc.store_scatter(offs, [idx], cur + 1)
                pos2[g // 8, pl.ds((g % 8) * 16, 16)] = cur
                dl2[g // 8, pl.ds((g % 8) * 16, 16)] = dv - b * 160
                return 0
            lax.fori_loop(0, g_n, body, 0)

            cps = []
            for i in range(16):
                cps.append(pltpu.async_copy(
                    sraw.at[pl.ds(i * 128, 128)], srcp_ref.at[pos2.at[i]],
                    sem))
                cps.append(pltpu.async_copy(
                    dl2.at[i], dstlp_ref.at[pos2.at[i]], sem2))
            for cp in cps:
                cp.wait()

    return k(src, dst, starts)


def _lin1_body(x_ref, w_ref, b_ref, d_ref, h_ref, ah_ref):
    i = pl.program_id(0)
    h = jnp.dot(x_ref[...], w_ref[...],
                preferred_element_type=jnp.float32) + b_ref[...]
    rows = lax.broadcasted_iota(jnp.int32, (256, 1), 0) + i * 256
    h = jnp.where(rows < N, h, 0.0)
    d = d_ref[...].reshape(256, 1)
    a = lax.rsqrt(d + 1.0)
    h_ref[...] = h
    ah_ref[...] = h * a


def _lin1(xp, w, b, deg2d):
    return pl.pallas_call(
        _lin1_body,
        grid=(NP // 256,),
        in_specs=[pl.BlockSpec((256, H), lambda i: (i, 0)),
                  pl.BlockSpec((H, H), lambda i: (0, 0)),
                  pl.BlockSpec((1, H), lambda i: (0, 0)),
                  pl.BlockSpec((1, 1, 256), lambda i: (i, 0, 0))],
        out_specs=[pl.BlockSpec((256, H), lambda i: (i, 0)),
                   pl.BlockSpec((256, H), lambda i: (i, 0))],
        out_shape=[jax.ShapeDtypeStruct((NP, H), jnp.float32),
                   jax.ShapeDtypeStruct((NP, H), jnp.float32)],
    )(xp, w, b, deg2d)


def _layer(h, ah, srcp, dstlp, starts):
    @functools.partial(
        pl.kernel,
        out_type=[jax.ShapeDtypeStruct((NP, H), jnp.float32),
                  jax.ShapeDtypeStruct((NP, H), jnp.float32),
                  jax.ShapeDtypeStruct((NP, H), jnp.float32)],
        mesh=_mesh(),
        compiler_params=pltpu.CompilerParams(needs_layout_passes=False),
        scratch_types=[
            pltpu.VMEM((16,), jnp.int32),
            pltpu.VMEM((16,), jnp.int32),
            pltpu.VMEM((16,), jnp.int32),
            pltpu.VMEM((K,), jnp.int32),
            pltpu.VMEM((K,), jnp.int32),
            pltpu.VMEM((K,), jnp.int32),
            pltpu.VMEM((K,), jnp.int32),
            pltpu.VMEM((K, H), jnp.float32),
            pltpu.VMEM((K, H), jnp.float32),
            pltpu.VMEM((168, H), jnp.float32),
            pltpu.VMEM((168, H), jnp.float32),
            pltpu.VMEM((64, H), jnp.float32),
            pltpu.VMEM_SHARED((5128, H), jnp.float32),
            pltpu.SemaphoreType.DMA,
            pltpu.SemaphoreType.DMA,
            pltpu.SemaphoreType.DMA,
        ],
    )
    def k(h_ref, ah_ref, srcp_ref, dstlp_ref, starts_ref,
          sum_ref, gcn_ref, mx_ref,
          t1, t2, t3, srcb, dlb, posb, dvm, rows_h, rows_a, accm, accg, zb,
          sum_sh, semg1, semg2, sems1):
        c = lax.axis_index("c")
        s = lax.axis_index("s")
        t = c * 16 + s
        lane = _i16()
        zf = jnp.zeros((16,), jnp.float32)
        ninf = jnp.full((16,), -jnp.inf, jnp.float32)

        pltpu.sync_copy(starts_ref.at[0, pl.ds(32 * t, 16)], t1)
        pltpu.sync_copy(starts_ref.at[0, pl.ds(32 * t + 16, 16)], t2)
        off3 = jnp.where(t == 31, 0, 32 * t + 32)
        pltpu.sync_copy(starts_ref.at[0, pl.ds(off3, 16)], t3)
        v1 = t1[:]
        v2 = t2[:]
        v3 = t3[:]
        e0 = v1[0]
        e1 = v2[0]
        e2 = jnp.where(t == 31, E, v3[0])

        def zz(i, _):
            for cc in range(8):
                zb[i, pl.ds(cc * 16, 16)] = zf
            return 0
        lax.fori_loop(0, 64, zz, 0)
        for i in range(5):
            pltpu.sync_copy(zb, sum_sh.at[pl.ds(320 * s + 64 * i, 64)])

        for r in range(2):
            es = e0 if r == 0 else e1
            ee = e1 if r == 0 else e2

            def za(i, _):
                for cc in range(8):
                    accm[i, pl.ds(cc * 16, 16)] = ninf
                    accg[i, pl.ds(cc * 16, 16)] = zf
                return 0
            lax.fori_loop(0, 168, za, 0)

            cb0 = jnp.bitwise_and(es, -8)
            nch = (ee - cb0 + K - 1) // K

            def chunk(f, _):
                cb = pl.multiple_of(cb0 + f * K, 8)
                pltpu.sync_copy(srcp_ref.at[pl.ds(cb, K)], srcb)
                pltpu.sync_copy(dstlp_ref.at[pl.ds(cb, K)], dlb)
                gh = pltpu.async_copy(h_ref.at[srcb], rows_h, semg1)
                ga = pltpu.async_copy(ah_ref.at[srcb], rows_a, semg2)
                for g in range(8):
                    dv = dlb[pl.ds(g * 16, 16)]
                    ge = cb + g * 16 + lane
                    valid = (ge >= es) & (ge < ee)
                    posb[pl.ds(g * 16, 16)] = jnp.where(
                        valid, dv + (320 * s + 160 * r), DUMP)
                    dvm[pl.ds(g * 16, 16)] = jnp.where(valid, dv, 160)
                gh.wait()
                ga.wait()
                cps = pltpu.async_copy(rows_h, sum_sh.at[posb], sems1,
                                       add=True)

                def mg(g, _):
                    dv2 = dvm[pl.ds(g * 16, 16)]
                    for j in range(16):
                        d = dv2[j]
                        for cc in range(8):
                            sl = pl.ds(cc * 16, 16)
                            rv = rows_h[g * 16 + j, sl]
                            accm[d, sl] = jnp.maximum(accm[d, sl], rv)
                            plsc.addupdate(accg.at[d, sl],
                                           rows_a[g * 16 + j, sl])
                    return 0
                lax.fori_loop(0, 8, mg, 0)
                cps.wait()
                return 0
            lax.fori_loop(0, nch, chunk, 0)
            b = 2 * t + r
            pltpu.sync_copy(accm.at[pl.ds(0, 160)],
                            mx_ref.at[pl.ds(b * 160, 160)])
            pltpu.sync_copy(accg.at[pl.ds(0, 160)],
                            gcn_ref.at[pl.ds(b * 160, 160)])

        pltpu.sync_copy(sum_sh.at[pl.ds(320 * s, 320)],
                        sum_ref.at[pl.ds(5120 * c + 320 * s, 320)])

    return k(h, ah, srcp, dstlp, starts)


def _passa_body(s_ref, g_ref, m_ref, h_ref, d_ref, w0_ref, w1_ref, w2_ref,
                w3_ref, bias_ref, pre_ref, st_ref):
    i = pl.program_id(0)
    d = d_ref[...].reshape(256, 1)
    invd = 1.0 / jnp.maximum(d, 1.0)
    a = lax.rsqrt(d + 1.0)
    invd1 = 1.0 / (d + 1.0)
    sm = s_ref[...]
    gc = g_ref[...]
    mx = m_ref[...]
    h = h_ref[...]
    mf = jnp.where(mx > NEG, mx, 0.0)
    gcn = a * gc + invd1 * h
    pre = (jnp.dot(sm, w0_ref[...], preferred_element_type=jnp.float32)
           + jnp.dot(invd * sm, w1_ref[...],
                     preferred_element_type=jnp.float32)
           + jnp.dot(mf, w2_ref[...], preferred_element_type=jnp.float32)
           + jnp.dot(gcn, w3_ref[...], preferred_element_type=jnp.float32)
           + bias_ref[...])
    rows = lax.broadcasted_iota(jnp.int32, (256, 1), 0) + i * 256
    pre = jnp.where(rows < N, pre, 0.0)
    pre_ref[...] = pre

    @pl.when(i == 0)
    def _():
        st_ref[...] = jnp.zeros((8, H), jnp.float32)

    st_ref[0:1, :] += jnp.sum(pre, axis=0, keepdims=True)
    st_ref[1:2, :] += jnp.sum(pre * pre, axis=0, keepdims=True)


def _passa(su, gc, mx, h, deg2d, w0, w1, w2, w3, bias):
    blk = lambda i: (i, 0)
    fixed = lambda i: (0, 0)
    return pl.pallas_call(
        _passa_body,
        grid=(NP // 256,),
        in_specs=[pl.BlockSpec((256, H), blk)] * 4 + [
            pl.BlockSpec((1, 1, 256), lambda i: (i, 0, 0)),
            pl.BlockSpec((H, H), fixed),
            pl.BlockSpec((H, H), fixed),
            pl.BlockSpec((H, H), fixed),
            pl.BlockSpec((H, H), fixed),
            pl.BlockSpec((1, H), fixed)],
        out_specs=[pl.BlockSpec((256, H), blk),
                   pl.BlockSpec((8, H), fixed)],
        out_shape=[jax.ShapeDtypeStruct((NP, H), jnp.float32),
                   jax.ShapeDtypeStruct((8, H), jnp.float32)],
    )(su, gc, mx, h, deg2d, w0, w1, w2, w3, bias)


def _passb_body(pre_ref, st_ref, gam_ref, bet_ref, d_ref, aw_ref, h_ref,
                ah_ref):
    i = pl.program_id(0)
    mu = st_ref[0:1, :] / float(N)
    ex2 = st_ref[1:2, :] / float(N)
    var = ex2 - mu * mu
    x = ((pre_ref[...] - mu) * lax.rsqrt(var + 1e-5) * gam_ref[...]
         + bet_ref[...])
    acts = [
        x,
        jnp.where(x > 0, x, jnp.exp(jnp.minimum(x, 0.0)) - 1.0),
        jax.nn.sigmoid(x),
        jnp.tanh(x),
        jnp.maximum(x, 0.0),
        jnp.clip(x, 0.0, 6.0),
        jnp.maximum(x, 0.0) + jnp.log(1.0 + jnp.exp(-jnp.abs(x))),
        jnp.where(x > 0, x, 0.01 * x),
    ]
    out = acts[0] * aw_ref[0]
    for j in range(1, 8):
        out = out + acts[j] * aw_ref[j]
    rows = lax.broadcasted_iota(jnp.int32, (256, 1), 0) + i * 256
    out = jnp.where(rows < N, out, 0.0)
    d = d_ref[...].reshape(256, 1)
    a = lax.rsqrt(d + 1.0)
    h_ref[...] = out
    ah_ref[...] = out * a


def _passb(pre, st, gam, bet, deg2d, aw):
    blk = lambda i: (i, 0)
    fixed = lambda i: (0, 0)
    return pl.pallas_call(
        _passb_body,
        grid=(NP // 256,),
        in_specs=[pl.BlockSpec((256, H), blk),
                  pl.BlockSpec((8, H), fixed),
                  pl.BlockSpec((1, H), fixed),
                  pl.BlockSpec((1, H), fixed),
                  pl.BlockSpec((1, 1, 256), lambda i: (i, 0, 0)),
                  pl.BlockSpec(memory_space=pltpu.SMEM)],
        out_specs=[pl.BlockSpec((256, H), blk),
                   pl.BlockSpec((256, H), blk)],
        out_shape=[jax.ShapeDtypeStruct((NP, H), jnp.float32),
                   jax.ShapeDtypeStruct((NP, H), jnp.float32)],
    )(pre, st, gam, bet, deg2d, aw)


def _readout(batch, h0, h1, h2, h3):
    @functools.partial(
        pl.kernel,
        out_type=[jax.ShapeDtypeStruct((NW, 2, 512), jnp.float32),
                  jax.ShapeDtypeStruct((NW, 2, 512), jnp.float32),
                  jax.ShapeDtypeStruct((NW, 2, 16), jnp.float32)],
        mesh=_mesh(),
        compiler_params=pltpu.CompilerParams(needs_layout_passes=False),
        scratch_types=[
            pltpu.VMEM((N,), jnp.int32),
            pltpu.VMEM((64, H), jnp.float32),
            pltpu.VMEM((2, 512), jnp.float32),
            pltpu.VMEM((2, 512), jnp.float32),
            pltpu.VMEM((2, 16), jnp.float32),
        ],
    )
    def k(b_ref, a0_ref, a1_ref, a2_ref, a3_ref, gs_ref, gm_ref, gc_ref,
          bb, rbuf, gs, gm, gcv):
        c = lax.axis_index("c")
        s = lax.axis_index("s")
        t = c * 16 + s
        g0 = 2 * t
        pltpu.sync_copy(b_ref, bb)
        zi = jnp.zeros((16,), jnp.int32)
        one = jnp.full((16,), 1, jnp.int32)

        def cnt_body(i, carry):
            c0, c1, c2 = carry
            bv = bb[pl.ds(i * 16, 16)]
            c0 = c0 + jnp.where(bv < g0, one, zi)
            c1 = c1 + jnp.where(bv < g0 + 1, one, zi)
            c2 = c2 + jnp.where(bv < g0 + 2, one, zi)
            return c0, c1, c2
        c0, c1, c2 = lax.fori_loop(0, N // 16, cnt_body, (zi, zi, zi))
        n0 = jnp.sum(c0)
        n1 = jnp.sum(c1)
        n2 = jnp.sum(c2)

        zf = jnp.zeros((16,), jnp.float32)
        ninf = jnp.full((16,), -jnp.inf, jnp.float32)

        for r in range(2):
            ns = n0 if r == 0 else n1
            ne = n1 if r == 0 else n2
            cnt = ne - ns
            gcv[r, :] = jnp.zeros((16,), jnp.float32) + cnt.astype(
                jnp.float32)
            cb0 = jnp.bitwise_and(ns, -8)
            for ai, aref in enumerate((a0_ref, a1_ref, a2_ref, a3_ref)):
                nch = (ne - cb0 + 63) // 64

                def chunk(f, carry):
                    cb = pl.multiple_of(cb0 + f * 64, 8)
                    pltpu.sync_copy(aref.at[pl.ds(cb, 64)], rbuf)

                    def row(j, cr):
                        valid = ((cb + j) >= ns) & ((cb + j) < ne)
                        out = []
                        for cc in range(8):
                            v = rbuf[j, pl.ds(cc * 16, 16)]
                            sa = cr[cc] + jnp.where(valid, v, zf)
                            ma = jnp.maximum(
                                cr[8 + cc], jnp.where(valid, v, ninf))
                            out.append((sa, ma))
                        return tuple(x[0] for x in out) + tuple(
                            x[1] for x in out)
                    return lax.fori_loop(0, 64, row, carry)
                init = tuple([zf] * 8 + [ninf] * 8)
                res = lax.fori_loop(0, nch, chunk, init)
                for cc in range(8):
                    gs[r, pl.ds(ai * 128 + cc * 16, 16)] = res[cc]
                    gm[r, pl.ds(ai * 128 + cc * 16, 16)] = res[8 + cc]
        pltpu.sync_copy(gs, gs_ref.at[t])
        pltpu.sync_copy(gm, gm_ref.at[t])
        pltpu.sync_copy(gcv, gc_ref.at[t])

    return k(batch, h0, h1, h2, h3)


def _head_body(gs_ref, gm_ref, gc_ref, w1_ref, w2_ref, b_ref, cw_ref,
               cb_ref, o_ref):
    cnt = gc_ref[...][:, 0:1]
    mean = gs_ref[...] / jnp.maximum(cnt, 1.0)
    mx = jnp.where(gm_ref[...] > NEG, gm_ref[...], 0.0)
    z = (jnp.dot(mean, w1_ref[...], preferred_element_type=jnp.float32)
         + jnp.dot(mx, w2_ref[...], preferred_element_type=jnp.float32)
         + b_ref[...])
    z = jnp.maximum(z, 0.0)
    o_ref[...] = jnp.dot(z, cw_ref[...],
                         preferred_element_type=jnp.float32) + cb_ref[...]


def _head(gs, gm, gc, w1, w2, b, cw, cb):
    return pl.pallas_call(
        _head_body,
        out_shape=jax.ShapeDtypeStruct((NG, 128), jnp.float32),
    )(gs, gm, gc, w1, w2, b, cw, cb)


def kernel(x, edge_index, batch, lin1_W, lin1_b, W_gnn, b_gnn, bn_gamma,
           bn_beta, na_w, act_w, lo_W, lo_b, cls_W, cls_b):
    xp = jnp.concatenate([x, jnp.zeros((NP - N, H), jnp.float32)], axis=0)
    wl = na_w[:, :, None, None] * W_gnn
    bias_l = jnp.sum(na_w[:, :, None] * b_gnn, axis=1)
    low1 = lo_W[:512]
    low2 = lo_W[512:]
    cwp = jnp.concatenate(
        [cls_W, jnp.zeros((H, 128 - OUT), jnp.float32)], axis=1)
    cbp = jnp.concatenate(
        [cls_b, jnp.zeros((128 - OUT,), jnp.float32)]).reshape(1, 128)

    src = edge_index[0]
    dst = edge_index[1]
    hist, degp = _p1(dst)
    starts, deg2d = _offsets(hist, degp)
    srcp, dstlp = _p2(src, dst, starts)
    h, ah = _lin1(xp, lin1_W, lin1_b.reshape(1, H), deg2d)
    hs = [h]
    for l in range(LAYERS):
        su, gcs, mx = _layer(h, ah, srcp, dstlp, starts)
        pre, st = _passa(su, gcs, mx, h, deg2d, wl[l, 0], wl[l, 1],
                         wl[l, 2], wl[l, 3], bias_l[l].reshape(1, H))
        h, ah = _passb(pre, st, bn_gamma[l].reshape(1, H),
                       bn_beta[l].reshape(1, H), deg2d, act_w[l])
        hs.append(h)
    gs, gm, gc = _readout(batch, hs[0], hs[1], hs[2], hs[3])
    gs = gs.reshape(NG, 512)
    gm = gm.reshape(NG, 512)
    gc = gc.reshape(NG, 16)
    logits = _head(gs, gm, gc, low1, low2, lo_b.reshape(1, H), cwp, cbp)
    return logits[:, :OUT]


# speedup vs baseline: 1.3765x; 1.3765x over previous
import functools

import jax
import jax.numpy as jnp
from jax import lax
from jax.experimental import pallas as pl
from jax.experimental.pallas import tpu as pltpu
from jax.experimental.pallas import tpu_sc as plsc

N = 10000
E = 320000
H = 128
LAYERS = 3
NG = 64
OUT = 10
NP = 10240
NBUK = 64
BR = 160
NW = 32
EC = E // NW
EP = E + 256
DUMP = 5120
K = 128
NEG = -3.0e38

_i16 = lambda: lax.iota(jnp.int32, 16)


def _mesh():
    return plsc.VectorSubcoreMesh(core_axis_name="c", subcore_axis_name="s")


def _p1(dst):
    @functools.partial(
        pl.kernel,
        out_type=[jax.ShapeDtypeStruct((NW, 1024), jnp.int32),
                  jax.ShapeDtypeStruct((2 * NP,), jnp.float32)],
        mesh=_mesh(),
        compiler_params=pltpu.CompilerParams(needs_layout_passes=False),
        scratch_types=[
            pltpu.VMEM((2048,), jnp.int32),
            pltpu.VMEM((16, 128), jnp.int32),
            pltpu.VMEM((128,), jnp.float32),
            pltpu.VMEM((1040,), jnp.int32),
            pltpu.VMEM((640,), jnp.float32),
            pltpu.VMEM_SHARED((NP + 16,), jnp.float32),
            pltpu.SemaphoreType.DMA,
        ],
    )
    def k(dst_ref, hist_ref, degp_ref, draw, d2, onesb, histv, zb, deg_sh,
          sem):
        c = lax.axis_index("c")
        s = lax.axis_index("s")
        wid = c * 16 + s
        zf = jnp.zeros((16,), jnp.float32)
        zi = jnp.zeros((16,), jnp.int32)
        one_i = jnp.full((16,), 1, jnp.int32)
        onef = jnp.full((16,), 1.0, jnp.float32)
        lane = _i16()

        def z1(i, _):
            histv[pl.ds(i * 16, 16)] = zi
            return 0
        lax.fori_loop(0, 65, z1, 0)

        def z2(i, _):
            zb[pl.ds(i * 16, 16)] = zf
            return 0
        lax.fori_loop(0, 40, z2, 0)
        for i in range(8):
            onesb[pl.ds(i * 16, 16)] = onef
        pltpu.sync_copy(zb, deg_sh.at[pl.ds(s * 640, 640)])

        @pl.when(s == 0)
        def _():
            zb16 = zb.at[pl.ds(0, 16)]
            pltpu.sync_copy(zb16, deg_sh.at[pl.ds(NP, 16)])

        plsc.subcore_barrier()

        for ch in range(5):
            sz = 2048 if ch < 4 else 1808
            g_n = sz // 16
            pltpu.sync_copy(dst_ref.at[pl.ds(wid * EC + ch * 2048, sz)],
                            draw.at[pl.ds(0, sz)])
            if ch == 4:
                dumpv = jnp.full((16,), NP, jnp.int32)
                def zt(i, _):
                    for cc in range(8):
                        d2[i, pl.ds(cc * 16, 16)] = dumpv
                    return 0
                lax.fori_loop(0, 16, zt, 0)

            def body(g, _):
                dv = draw[pl.ds(g * 16, 16)]
                b = jnp.right_shift(dv * 13108, 21)
                idx = b * 16 + lane
                plsc.addupdate_scatter(histv, [idx], one_i)
                d2[g // 8, pl.ds((g % 8) * 16, 16)] = dv
                return 0
            lax.fori_loop(0, g_n, body, 0)

            cps = [pltpu.async_copy(onesb, deg_sh.at[d2.at[i]], sem,
                                    add=True) for i in range(16)]
            for cp in cps:
                cp.wait()

        plsc.subcore_barrier()
        pltpu.sync_copy(histv.at[pl.ds(0, 1024)], hist_ref.at[wid])
        pltpu.sync_copy(deg_sh.at[pl.ds(s * 640, 640)],
                        degp_ref.at[pl.ds(c * NP + s * 640, 640)])

    return k(dst)


def _off_body(hist_ref, degp_ref, starts_ref, deg_ref, a_ref):
    cnt = hist_ref[...].astype(jnp.float32)
    ii = lax.broadcasted_iota(jnp.int32, (1024, 1024), 0)
    jj = lax.broadcasted_iota(jnp.int32, (1024, 1024), 1)
    m_l = ((ii // 16 == jj // 16) & ((ii % 16) < (jj % 16))).astype(
        jnp.float32)
    lstart = jnp.dot(cnt, m_l, preferred_element_type=jnp.float32, precision=lax.Precision.HIGHEST)
    r64 = lax.broadcasted_iota(jnp.int32, (1024, 64), 0) // 16
    c64 = lax.broadcasted_iota(jnp.int32, (1024, 64), 1)
    msum = (r64 == c64).astype(jnp.float32)
    tbc = jnp.dot(cnt, msum, preferred_element_type=jnp.float32, precision=lax.Precision.HIGHEST)
    aw = lax.broadcasted_iota(jnp.int32, (32, 32), 1)
    ar = lax.broadcasted_iota(jnp.int32, (32, 32), 0)
    a32 = (aw < ar).astype(jnp.float32)
    wstart = jnp.dot(a32, tbc, preferred_element_type=jnp.float32, precision=lax.Precision.HIGHEST)
    ones32 = jnp.ones((1, 32), jnp.float32)
    bcnt = jnp.dot(ones32, tbc, preferred_element_type=jnp.float32, precision=lax.Precision.HIGHEST)
    li = lax.broadcasted_iota(jnp.int32, (64, 64), 0)
    lj = lax.broadcasted_iota(jnp.int32, (64, 64), 1)
    lt64 = (li < lj).astype(jnp.float32)
    bs = jnp.dot(bcnt, lt64, preferred_element_type=jnp.float32, precision=lax.Precision.HIGHEST)
    e64r = lax.broadcasted_iota(jnp.int32, (64, 1024), 0)
    e64c = lax.broadcasted_iota(jnp.int32, (64, 1024), 1) // 16
    e64 = (e64r == e64c).astype(jnp.float32)
    bs_e = jnp.dot(bs, e64, preferred_element_type=jnp.float32, precision=lax.Precision.HIGHEST)
    ws_e = jnp.dot(wstart, e64, preferred_element_type=jnp.float32, precision=lax.Precision.HIGHEST)
    starts_ref[...] = (bs_e + ws_e + lstart).astype(jnp.int32)
    dp = degp_ref[...].reshape(2, NP)
    degsum = dp[0:1, :] + dp[1:2, :]
    deg_ref[...] = degsum.reshape(NP // 256, 1, 256)
    a_ref[...] = lax.rsqrt(degsum + 1.0)


def _offsets(hist, degp):
    return pl.pallas_call(
        _off_body,
        out_shape=[jax.ShapeDtypeStruct((NW, 1024), jnp.int32),
                   jax.ShapeDtypeStruct((NP // 256, 1, 256), jnp.float32),
                   jax.ShapeDtypeStruct((1, NP), jnp.float32)],
    )(hist, degp)


def _p2(src, dst, starts):
    @functools.partial(
        pl.kernel,
        out_type=jax.ShapeDtypeStruct((EP,), jnp.int32),
        mesh=_mesh(),
        compiler_params=pltpu.CompilerParams(needs_layout_passes=False),
        scratch_types=[
            pltpu.VMEM((1040,), jnp.int32),
            pltpu.VMEM((2048,), jnp.int32),
            pltpu.VMEM((2048,), jnp.int32),
            pltpu.VMEM((16, 128), jnp.int32),
            pltpu.VMEM((16, 128), jnp.int32),
            pltpu.VMEM((128,), jnp.int32),
            pltpu.SemaphoreType.DMA,
        ],
    )
    def k(src_ref, dst_ref, starts_ref, pkp_ref, offs, sraw,
          draw, pos2, pk2, padv, sem):
        c = lax.axis_index("c")
        s = lax.axis_index("s")
        wid = c * 16 + s
        lane = _i16()

        @pl.when(wid == 0)
        def _():
            pv = jnp.full((16,), N * 256, jnp.int32)
            for i in range(8):
                padv[pl.ds(i * 16, 16)] = pv
            pltpu.sync_copy(padv, pkp_ref.at[pl.ds(E, 128)])

        pltpu.sync_copy(starts_ref.at[wid], offs.at[pl.ds(0, 1024)])

        for ch in range(5):
            sz = 2048 if ch < 4 else 1808
            g_n = sz // 16
            base = wid * EC + ch * 2048
            pltpu.sync_copy(src_ref.at[pl.ds(base, sz)],
                            sraw.at[pl.ds(0, sz)])
            pltpu.sync_copy(dst_ref.at[pl.ds(base, sz)],
                            draw.at[pl.ds(0, sz)])
            if ch == 4:
                def zt(i, _):
                    dump = jnp.full((16,), E + 128, jnp.int32) + lane
                    for cc in range(8):
                        pos2[i, pl.ds(cc * 16, 16)] = dump
                    return 0
                lax.fori_loop(0, 16, zt, 0)

            def body(g, _):
                dv = draw[pl.ds(g * 16, 16)]
                sv = sraw[pl.ds(g * 16, 16)]
                b = jnp.right_shift(dv * 13108, 21)
                idx = b * 16 + lane
                cur = plsc.load_gather(offs, [idx])
                plsc.store_scatter(offs, [idx], cur + 1)
                pos2[g // 8, pl.ds((g % 8) * 16, 16)] = cur
                pk2[g // 8, pl.ds((g % 8) * 16, 16)] = (
                    sv * 256 + (dv - b * 160))
                return 0
            lax.fori_loop(0, g_n, body, 0)

            cps = [pltpu.async_copy(pk2.at[i], pkp_ref.at[pos2.at[i]], sem)
                   for i in range(16)]
            for cp in cps:
                cp.wait()

    return k(src, dst, starts)


def _lin1_body(x_ref, w_ref, b_ref, h_ref):
    i = pl.program_id(0)
    h = jnp.dot(x_ref[...], w_ref[...],
                preferred_element_type=jnp.float32) + b_ref[...]
    rows = lax.broadcasted_iota(jnp.int32, (256, 1), 0) + i * 256
    h_ref[...] = jnp.where(rows < N, h, 0.0)


def _lin1(xp, w, b):
    return pl.pallas_call(
        _lin1_body,
        grid=(NP // 256,),
        in_specs=[pl.BlockSpec((256, H), lambda i: (i, 0)),
                  pl.BlockSpec((H, H), lambda i: (0, 0)),
                  pl.BlockSpec((1, H), lambda i: (0, 0))],
        out_specs=pl.BlockSpec((256, H), lambda i: (i, 0)),
        out_shape=jax.ShapeDtypeStruct((NP, H), jnp.float32),
    )(xp, w, b)


def _layer(h, pkp, starts, a1d):
    @functools.partial(
        pl.kernel,
        out_type=[jax.ShapeDtypeStruct((NP, H), jnp.float32),
                  jax.ShapeDtypeStruct((NP, H), jnp.float32),
                  jax.ShapeDtypeStruct((NP, H), jnp.float32)],
        mesh=_mesh(),
        compiler_params=pltpu.CompilerParams(needs_layout_passes=False),
        scratch_types=[
            pltpu.VMEM((16,), jnp.int32),
            pltpu.VMEM((16,), jnp.int32),
            pltpu.VMEM((16,), jnp.int32),
            pltpu.VMEM((K,), jnp.int32),
            pltpu.VMEM((K,), jnp.int32),
            pltpu.VMEM((K,), jnp.int32),
            pltpu.VMEM((K,), jnp.int32),
            pltpu.VMEM((K,), jnp.float32),
            pltpu.VMEM((K, H), jnp.float32),
            pltpu.VMEM((168, H), jnp.float32),
            pltpu.VMEM((168, H), jnp.float32),
            pltpu.VMEM((64, H), jnp.float32),
            pltpu.VMEM_SHARED((5128, H), jnp.float32),
            pltpu.SemaphoreType.DMA,
            pltpu.SemaphoreType.DMA,
            pltpu.SemaphoreType.DMA,
        ],
    )
    def k(h_ref, pkp_ref, starts_ref, a_ref,
          sum_ref, gcn_ref, mx_ref,
          t1, t2, t3, pkb, srcb, posb, dvm, av, rows_h, accm, accg, zb,
          sum_sh, semg1, semg2, sems1):
        c = lax.axis_index("c")
        s = lax.axis_index("s")
        t = c * 16 + s
        lane = _i16()
        zf = jnp.zeros((16,), jnp.float32)
        ninf = jnp.full((16,), -jnp.inf, jnp.float32)

        pltpu.sync_copy(starts_ref.at[0, pl.ds(32 * t, 16)], t1)
        pltpu.sync_copy(starts_ref.at[0, pl.ds(32 * t + 16, 16)], t2)
        off3 = jnp.where(t == 31, 0, 32 * t + 32)
        pltpu.sync_copy(starts_ref.at[0, pl.ds(off3, 16)], t3)
        v1 = t1[:]
        v2 = t2[:]
        v3 = t3[:]
        e0 = v1[0]
        e1 = v2[0]
        e2 = jnp.where(t == 31, E, v3[0])

        def zz(i, _):
            for cc in range(8):
                zb[i, pl.ds(cc * 16, 16)] = zf
            return 0
        lax.fori_loop(0, 64, zz, 0)
        for i in range(5):
            pltpu.sync_copy(zb, sum_sh.at[pl.ds(320 * s + 64 * i, 64)])

        for r in range(2):
            es = e0 if r == 0 else e1
            ee = e1 if r == 0 else e2

            def za(i, _):
                for cc in range(8):
                    accm[i, pl.ds(cc * 16, 16)] = ninf
                    accg[i, pl.ds(cc * 16, 16)] = zf
                return 0
            lax.fori_loop(0, 168, za, 0)

            cb0 = jnp.bitwise_and(es, -8)
            nch = (ee - cb0 + K - 1) // K

            def chunk(f, _):
                cb = pl.multiple_of(cb0 + f * K, 8)
                pltpu.sync_copy(pkp_ref.at[pl.ds(cb, K)], pkb)
                for g in range(8):
                    pv = pkb[pl.ds(g * 16, 16)]
                    dv = jnp.bitwise_and(pv, 255)
                    srcb[pl.ds(g * 16, 16)] = jnp.right_shift(pv, 8)
                    ge = cb + g * 16 + lane
                    valid = (ge >= es) & (ge < ee)
                    posb[pl.ds(g * 16, 16)] = jnp.where(
                        valid, dv + (320 * s + 160 * r), DUMP)
                    dvm[pl.ds(g * 16, 16)] = jnp.where(valid, dv, 160)
                gh = pltpu.async_copy(h_ref.at[srcb], rows_h, semg1)
                ga = pltpu.async_copy(a_ref.at[srcb], av, semg2)
                gh.wait()
                ga.wait()
                cps = pltpu.async_copy(rows_h, sum_sh.at[posb], sems1,
                                       add=True)

                def mg(g, _):
                    dv2 = dvm[pl.ds(g * 16, 16)]
                    avv = av[pl.ds(g * 16, 16)]
                    for j in range(16):
                        d = dv2[j]
                        a_s = avv[j]
                        for cc in range(8):
                            sl = pl.ds(cc * 16, 16)
                            rv = rows_h[g * 16 + j, sl]
                            accm[d, sl] = jnp.maximum(accm[d, sl], rv)
                            plsc.addupdate(accg.at[d, sl], rv * a_s)
                    return 0
                lax.fori_loop(0, 8, mg, 0)
                cps.wait()
                return 0
            lax.fori_loop(0, nch, chunk, 0)
            b = 2 * t + r
            pltpu.sync_copy(accm.at[pl.ds(0, 160)],
                            mx_ref.at[pl.ds(b * 160, 160)])
            pltpu.sync_copy(accg.at[pl.ds(0, 160)],
                            gcn_ref.at[pl.ds(b * 160, 160)])

        pltpu.sync_copy(sum_sh.at[pl.ds(320 * s, 320)],
                        sum_ref.at[pl.ds(5120 * c + 320 * s, 320)])

    return k(h, pkp, starts, a1d)


def _passa_body(s_ref, g_ref, m_ref, h_ref, d_ref, w0_ref, w1_ref, w2_ref,
                w3_ref, bias_ref, pre_ref, st_ref):
    i = pl.program_id(0)
    d = d_ref[...].reshape(256, 1)
    invd = 1.0 / jnp.maximum(d, 1.0)
    a = lax.rsqrt(d + 1.0)
    invd1 = 1.0 / (d + 1.0)
    sm = s_ref[...]
    gc = g_ref[...]
    mx = m_ref[...]
    h = h_ref[...]
    mf = jnp.where(mx > NEG, mx, 0.0)
    gcn = a * gc + invd1 * h
    pre = (jnp.dot(sm, w0_ref[...], preferred_element_type=jnp.float32)
           + jnp.dot(invd * sm, w1_ref[...],
                     preferred_element_type=jnp.float32)
           + jnp.dot(mf, w2_ref[...], preferred_element_type=jnp.float32)
           + jnp.dot(gcn, w3_ref[...], preferred_element_type=jnp.float32)
           + bias_ref[...])
    rows = lax.broadcasted_iota(jnp.int32, (256, 1), 0) + i * 256
    pre = jnp.where(rows < N, pre, 0.0)
    pre_ref[...] = pre

    @pl.when(i == 0)
    def _():
        st_ref[...] = jnp.zeros((8, H), jnp.float32)

    st_ref[0:1, :] += jnp.sum(pre, axis=0, keepdims=True)
    st_ref[1:2, :] += jnp.sum(pre * pre, axis=0, keepdims=True)


def _passa(su, gc, mx, h, deg2d, w0, w1, w2, w3, bias):
    blk = lambda i: (i, 0)
    fixed = lambda i: (0, 0)
    return pl.pallas_call(
        _passa_body,
        grid=(NP // 256,),
        in_specs=[pl.BlockSpec((256, H), blk)] * 4 + [
            pl.BlockSpec((1, 1, 256), lambda i: (i, 0, 0)),
            pl.BlockSpec((H, H), fixed),
            pl.BlockSpec((H, H), fixed),
            pl.BlockSpec((H, H), fixed),
            pl.BlockSpec((H, H), fixed),
            pl.BlockSpec((1, H), fixed)],
        out_specs=[pl.BlockSpec((256, H), blk),
                   pl.BlockSpec((8, H), fixed)],
        out_shape=[jax.ShapeDtypeStruct((NP, H), jnp.float32),
                   jax.ShapeDtypeStruct((8, H), jnp.float32)],
    )(su, gc, mx, h, deg2d, w0, w1, w2, w3, bias)


def _passb_body(pre_ref, st_ref, gam_ref, bet_ref, aw_ref, h_ref):
    i = pl.program_id(0)
    mu = st_ref[0:1, :] / float(N)
    ex2 = st_ref[1:2, :] / float(N)
    var = ex2 - mu * mu
    x = ((pre_ref[...] - mu) * lax.rsqrt(var + 1e-5) * gam_ref[...]
         + bet_ref[...])
    acts = [
        x,
        jnp.where(x > 0, x, jnp.exp(jnp.minimum(x, 0.0)) - 1.0),
        jax.nn.sigmoid(x),
        jnp.tanh(x),
        jnp.maximum(x, 0.0),
        jnp.clip(x, 0.0, 6.0),
        jnp.maximum(x, 0.0) + jnp.log(1.0 + jnp.exp(-jnp.abs(x))),
        jnp.where(x > 0, x, 0.01 * x),
    ]
    out = acts[0] * aw_ref[0]
    for j in range(1, 8):
        out = out + acts[j] * aw_ref[j]
    rows = lax.broadcasted_iota(jnp.int32, (256, 1), 0) + i * 256
    h_ref[...] = jnp.where(rows < N, out, 0.0)


def _passb(pre, st, gam, bet, aw):
    blk = lambda i: (i, 0)
    fixed = lambda i: (0, 0)
    return pl.pallas_call(
        _passb_body,
        grid=(NP // 256,),
        in_specs=[pl.BlockSpec((256, H), blk),
                  pl.BlockSpec((8, H), fixed),
                  pl.BlockSpec((1, H), fixed),
                  pl.BlockSpec((1, H), fixed),
                  pl.BlockSpec(memory_space=pltpu.SMEM)],
        out_specs=pl.BlockSpec((256, H), blk),
        out_shape=jax.ShapeDtypeStruct((NP, H), jnp.float32),
    )(pre, st, gam, bet, aw)


def _readout(batch, h0, h1, h2, h3):
    @functools.partial(
        pl.kernel,
        out_type=[jax.ShapeDtypeStruct((NW, 2, 512), jnp.float32),
                  jax.ShapeDtypeStruct((NW, 2, 512), jnp.float32),
                  jax.ShapeDtypeStruct((NW, 2, 16), jnp.float32)],
        mesh=_mesh(),
        compiler_params=pltpu.CompilerParams(needs_layout_passes=False),
        scratch_types=[
            pltpu.VMEM((N,), jnp.int32),
            pltpu.VMEM((64, H), jnp.float32),
            pltpu.VMEM((2, 512), jnp.float32),
            pltpu.VMEM((2, 512), jnp.float32),
            pltpu.VMEM((2, 16), jnp.float32),
        ],
    )
    def k(b_ref, a0_ref, a1_ref, a2_ref, a3_ref, gs_ref, gm_ref, gc_ref,
          bb, rbuf, gs, gm, gcv):
        c = lax.axis_index("c")
        s = lax.axis_index("s")
        t = c * 16 + s
        g0 = 2 * t
        pltpu.sync_copy(b_ref, bb)
        zi = jnp.zeros((16,), jnp.int32)
        one = jnp.full((16,), 1, jnp.int32)

        def cnt_body(i, carry):
            c0, c1, c2 = carry
            bv = bb[pl.ds(i * 16, 16)]
            c0 = c0 + jnp.where(bv < g0, one, zi)
            c1 = c1 + jnp.where(bv < g0 + 1, one, zi)
            c2 = c2 + jnp.where(bv < g0 + 2, one, zi)
            return c0, c1, c2
        c0, c1, c2 = lax.fori_loop(0, N // 16, cnt_body, (zi, zi, zi))
        n0 = jnp.sum(c0)
        n1 = jnp.sum(c1)
        n2 = jnp.sum(c2)

        zf = jnp.zeros((16,), jnp.float32)
        ninf = jnp.full((16,), -jnp.inf, jnp.float32)

        for r in range(2):
            ns = n0 if r == 0 else n1
            ne = n1 if r == 0 else n2
            cnt = ne - ns
            gcv[r, :] = jnp.zeros((16,), jnp.float32) + cnt.astype(
                jnp.float32)
            cb0 = jnp.bitwise_and(ns, -8)
            for ai, aref in enumerate((a0_ref, a1_ref, a2_ref, a3_ref)):
                nch = (ne - cb0 + 63) // 64

                def chunk(f, carry):
                    cb = pl.multiple_of(cb0 + f * 64, 8)
                    pltpu.sync_copy(aref.at[pl.ds(cb, 64)], rbuf)

                    def row(j, cr):
                        valid = ((cb + j) >= ns) & ((cb + j) < ne)
                        out = []
                        for cc in range(8):
                            v = rbuf[j, pl.ds(cc * 16, 16)]
                            sa = cr[cc] + jnp.where(valid, v, zf)
                            ma = jnp.maximum(
                                cr[8 + cc], jnp.where(valid, v, ninf))
                            out.append((sa, ma))
                        return tuple(x[0] for x in out) + tuple(
                            x[1] for x in out)
                    return lax.fori_loop(0, 64, row, carry)
                init = tuple([zf] * 8 + [ninf] * 8)
                res = lax.fori_loop(0, nch, chunk, init)
                for cc in range(8):
                    gs[r, pl.ds(ai * 128 + cc * 16, 16)] = res[cc]
                    gm[r, pl.ds(ai * 128 + cc * 16, 16)] = res[8 + cc]
        pltpu.sync_copy(gs, gs_ref.at[t])
        pltpu.sync_copy(gm, gm_ref.at[t])
        pltpu.sync_copy(gcv, gc_ref.at[t])

    return k(batch, h0, h1, h2, h3)


def _head_body(gs_ref, gm_ref, gc_ref, w1_ref, w2_ref, b_ref, cw_ref,
               cb_ref, o_ref):
    cnt = gc_ref[...][:, 0:1]
    mean = gs_ref[...] / jnp.maximum(cnt, 1.0)
    mx = jnp.where(gm_ref[...] > NEG, gm_ref[...], 0.0)
    z = (jnp.dot(mean, w1_ref[...], preferred_element_type=jnp.float32)
         + jnp.dot(mx, w2_ref[...], preferred_element_type=jnp.float32)
         + b_ref[...])
    z = jnp.maximum(z, 0.0)
    o_ref[...] = jnp.dot(z, cw_ref[...],
                         preferred_element_type=jnp.float32) + cb_ref[...]


def _head(gs, gm, gc, w1, w2, b, cw, cb):
    return pl.pallas_call(
        _head_body,
        out_shape=jax.ShapeDtypeStruct((NG, 128), jnp.float32),
    )(gs, gm, gc, w1, w2, b, cw, cb)


def kernel(x, edge_index, batch, lin1_W, lin1_b, W_gnn, b_gnn, bn_gamma,
           bn_beta, na_w, act_w, lo_W, lo_b, cls_W, cls_b):
    xp = jnp.concatenate([x, jnp.zeros((NP - N, H), jnp.float32)], axis=0)
    wl = na_w[:, :, None, None] * W_gnn
    bias_l = jnp.sum(na_w[:, :, None] * b_gnn, axis=1)
    low1 = lo_W[:512]
    low2 = lo_W[512:]
    cwp = jnp.concatenate(
        [cls_W, jnp.zeros((H, 128 - OUT), jnp.float32)], axis=1)
    cbp = jnp.concatenate(
        [cls_b, jnp.zeros((128 - OUT,), jnp.float32)]).reshape(1, 128)

    src = edge_index[0]
    dst = edge_index[1]
    hist, degp = _p1(dst)
    starts, deg2d, a1d = _offsets(hist, degp)
    a1d = a1d.reshape(NP)
    pkp = _p2(src, dst, starts)
    h = _lin1(xp, lin1_W, lin1_b.reshape(1, H))
    hs = [h]
    for l in range(LAYERS):
        su, gcs, mx = _layer(h, pkp, starts, a1d)
        pre, st = _passa(su, gcs, mx, h, deg2d, wl[l, 0], wl[l, 1],
                         wl[l, 2], wl[l, 3], bias_l[l].reshape(1, H))
        h = _passb(pre, st, bn_gamma[l].reshape(1, H),
                   bn_beta[l].reshape(1, H), act_w[l])
        hs.append(h)
    gs, gm, gc = _readout(batch, hs[0], hs[1], hs[2], hs[3])
    gs = gs.reshape(NG, 512)
    gm = gm.reshape(NG, 512)
    gc = gc.reshape(NG, 16)
    logits = _head(gs, gm, gc, low1, low2, lo_b.reshape(1, H), cwp, cbp)
    return logits[:, :OUT]
